# baseline (speedup 1.0000x reference)
import dataclasses
import functools

import jax
import jax.numpy as jnp
import numpy as np
from jax import lax
from jax.experimental import pallas as pl
from jax.experimental.pallas import tpu as pltpu
from jax.experimental.pallas import tpu_sc as plsc

B = 4096
L = 200
D = 128
DW = D // 2
OUT = 2
NC = 2
NS = 16
NW = NC * NS
BPW = B // NW
SP1 = 104
SP2 = L - SP1
LANES = 16
NG = DW // LANES
UNROLL = 8


def _pool_sums(x_flat, table_i32):
    mesh = plsc.VectorSubcoreMesh(core_axis_name="c", subcore_axis_name="s")
    cp = pltpu.CompilerParams()
    if "needs_layout_passes" in pltpu.CompilerParams.__dataclass_fields__:
        cp = dataclasses.replace(cp, needs_layout_passes=False)
    if "use_tc_tiling_on_sc" in pltpu.CompilerParams.__dataclass_fields__:
        cp = dataclasses.replace(cp, use_tc_tiling_on_sc=False)

    @functools.partial(
        pl.kernel,
        out_type=jax.ShapeDtypeStruct((B, D), jnp.float32),
        mesh=mesh,
        compiler_params=cp,
        scratch_types=[
            pltpu.VMEM((BPW * L,), jnp.int32),
            pltpu.VMEM((2, L, DW), jnp.int32),
            pltpu.VMEM((BPW, D), jnp.float32),
            pltpu.SemaphoreType.DMA,
            pltpu.SemaphoreType.DMA,
        ],
    )
    def k(x_hbm, table_hbm, out_hbm, idx_v, rows_v, acc_v, sem0, sem1):
        wid = lax.axis_index("s") * NC + lax.axis_index("c")
        base = wid * BPW
        pltpu.sync_copy(x_hbm.at[pl.ds(base * L, BPW * L)], idx_v)
        sems = (sem0, sem1)
        mask_hi = jnp.full((LANES,), -65536, jnp.int32)

        def start(r, buf):
            off = r * L
            pltpu.async_copy(
                table_hbm.at[idx_v.at[pl.ds(off, SP1)]],
                rows_v.at[buf, pl.ds(0, SP1)], sems[buf])
            pltpu.async_copy(
                table_hbm.at[idx_v.at[pl.ds(off + SP1, SP2)]],
                rows_v.at[buf, pl.ds(SP1, SP2)], sems[buf])

        def wait(buf):
            pltpu.make_async_copy(
                table_hbm.at[pl.ds(0, L)], rows_v.at[buf], sems[buf]).wait()

        def process(r, buf):
            rv = rows_v.at[buf]

            def body(i, accs):
                t0 = i * UNROLL
                for u in range(UNROLL):
                    new = []
                    for c in range(NG):
                        w = rv[t0 + u, pl.ds(c * LANES, LANES)]
                        lo = plsc.bitcast(w << 16, jnp.float32)
                        hi = plsc.bitcast(w & mask_hi, jnp.float32)
                        new.append(accs[2 * c] + lo)
                        new.append(accs[2 * c + 1] + hi)
                    accs = tuple(new)
                return accs

            accs = lax.fori_loop(
                0, L // UNROLL, body,
                tuple(jnp.zeros((LANES,), jnp.float32) for _ in range(2 * NG)))
            for c in range(NG):
                acc_v[r, pl.ds(2 * c * LANES, LANES)] = accs[2 * c]
                acc_v[r, pl.ds((2 * c + 1) * LANES, LANES)] = accs[2 * c + 1]

        start(0, 0)
        start(1, 1)

        @pl.loop(0, BPW - 2, step=2)
        def _(i):
            wait(0)
            process(i, 0)
            start(i + 2, 0)
            wait(1)
            process(i + 1, 1)
            start(i + 3, 1)

        wait(0)
        process(BPW - 2, 0)
        wait(1)
        process(BPW - 1, 1)

        pltpu.sync_copy(acc_v, out_hbm.at[pl.ds(base, BPW)])

    return k(x_flat, table_i32)


def _head(sums, length2d, w_pad, b_pad):
    BLK = 512

    def body(p_ref, l_ref, w_ref, b_ref, o_ref):
        p = p_ref[...] / l_ref[...]
        z = jnp.dot(p, w_ref[...], preferred_element_type=jnp.float32)
        o_ref[...] = 1.0 / (1.0 + jnp.exp(-(z + b_ref[...])))

    return pl.pallas_call(
        body,
        grid=(B // BLK,),
        in_specs=[
            pl.BlockSpec((BLK, D), lambda i: (i, 0)),
            pl.BlockSpec((BLK, 1), lambda i: (i, 0)),
            pl.BlockSpec((D, D), lambda i: (0, 0)),
            pl.BlockSpec((1, D), lambda i: (0, 0)),
        ],
        out_specs=pl.BlockSpec((BLK, D), lambda i: (i, 0)),
        out_shape=jax.ShapeDtypeStruct((B, D), jnp.float32),
    )(sums, length2d, w_pad, b_pad)


PERM = np.empty((D,), np.int32)
for _c in range(NG):
    for _j in range(LANES):
        PERM[32 * _c + _j] = 32 * _c + 2 * _j
        PERM[32 * _c + LANES + _j] = 32 * _c + 2 * _j + 1


def kernel(x, length, embed_table, W, b):
    x_flat = x.reshape(-1)
    table_bf16 = embed_table.astype(jnp.bfloat16)
    table_i32 = lax.bitcast_convert_type(
        table_bf16.reshape(-1, DW, 2), jnp.int32)
    sums = _pool_sums(x_flat, table_i32)
    w_pad = jnp.zeros((D, D), jnp.float32).at[:, :OUT].set(W.T)
    w_perm = w_pad[PERM, :]
    b_pad = jnp.zeros((1, D), jnp.float32).at[0, :OUT].set(b)
    out = _head(sums, length.reshape(B, 1), w_perm, b_pad)
    return out[:, :OUT]


# speedup vs baseline: 2.5000x; 2.5000x over previous
import dataclasses
import functools

import jax
import jax.numpy as jnp
import numpy as np
from jax import lax
from jax.experimental import pallas as pl
from jax.experimental.pallas import tpu as pltpu
from jax.experimental.pallas import tpu_sc as plsc

B = 4096
L = 200
D = 128
DW = D // 2
OUT = 2
NC = 2
NS = 16
NW = NC * NS
BPW = B // NW
SP1 = 104
SP2 = L - SP1
LANES = 16
NG = DW // LANES
UNROLL = 8


def _pool_sums(x_flat, table_i32):
    mesh = plsc.VectorSubcoreMesh(core_axis_name="c", subcore_axis_name="s")
    cp = pltpu.CompilerParams()
    if "needs_layout_passes" in pltpu.CompilerParams.__dataclass_fields__:
        cp = dataclasses.replace(cp, needs_layout_passes=False)
    if "use_tc_tiling_on_sc" in pltpu.CompilerParams.__dataclass_fields__:
        cp = dataclasses.replace(cp, use_tc_tiling_on_sc=False)

    @functools.partial(
        pl.kernel,
        out_type=jax.ShapeDtypeStruct((B, D), jnp.float32),
        mesh=mesh,
        compiler_params=cp,
        scratch_types=[
            pltpu.VMEM((BPW * L,), jnp.int32),
            pltpu.VMEM((2, L, DW), jnp.int32),
            pltpu.VMEM((BPW, D), jnp.float32),
            pltpu.SemaphoreType.DMA,
            pltpu.SemaphoreType.DMA,
        ],
    )
    def k(x_hbm, table_hbm, out_hbm, idx_v, rows_v, acc_v, sem0, sem1):
        wid = lax.axis_index("s") * NC + lax.axis_index("c")
        base = wid * BPW
        pltpu.sync_copy(x_hbm.at[pl.ds(base * L, BPW * L)], idx_v)
        sems = (sem0, sem1)
        mask_hi = jnp.full((LANES,), -65536, jnp.int32)

        def start(r, buf):
            off = r * L
            pltpu.async_copy(
                table_hbm.at[idx_v.at[pl.ds(off, SP1)]],
                rows_v.at[buf, pl.ds(0, SP1)], sems[buf])
            pltpu.async_copy(
                table_hbm.at[idx_v.at[pl.ds(off + SP1, SP2)]],
                rows_v.at[buf, pl.ds(SP1, SP2)], sems[buf])

        def wait(buf):
            pltpu.make_async_copy(
                table_hbm.at[pl.ds(0, L)], rows_v.at[buf], sems[buf]).wait()

        def process(r, buf):
            rv = rows_v.at[buf]

            def body(i, accs):
                t0 = i * UNROLL
                for u in range(UNROLL):
                    new = []
                    for c in range(NG):
                        w = rv[t0 + u, pl.ds(c * LANES, LANES)]
                        lo = plsc.bitcast(w << 16, jnp.float32)
                        hi = plsc.bitcast(w & mask_hi, jnp.float32)
                        new.append(accs[2 * c] + lo)
                        new.append(accs[2 * c + 1] + hi)
                    accs = tuple(new)
                return accs

            accs = lax.fori_loop(
                0, L // UNROLL, body,
                tuple(jnp.zeros((LANES,), jnp.float32) for _ in range(2 * NG)))
            for c in range(NG):
                acc_v[r, pl.ds(2 * c * LANES, LANES)] = accs[2 * c]
                acc_v[r, pl.ds((2 * c + 1) * LANES, LANES)] = accs[2 * c + 1]

        start(0, 0)
        start(1, 1)

        @pl.loop(0, BPW - 2, step=2)
        def _(i):
            wait(0)
            process(i, 0)
            start(i + 2, 0)
            wait(1)
            process(i + 1, 1)
            start(i + 3, 1)

        wait(0)
        process(BPW - 2, 0)
        wait(1)
        process(BPW - 1, 1)

        pltpu.sync_copy(acc_v, out_hbm.at[pl.ds(base, BPW)])

    return k(x_flat, table_i32)


def _head(sums, length2d, w_pad, b_pad):
    BLK = 512

    def body(p_ref, l_ref, w_ref, b_ref, o_ref):
        p = p_ref[...] / l_ref[...]
        z = jnp.dot(p, w_ref[...], preferred_element_type=jnp.float32)
        o_ref[...] = 1.0 / (1.0 + jnp.exp(-(z + b_ref[...])))

    return pl.pallas_call(
        body,
        grid=(B // BLK,),
        in_specs=[
            pl.BlockSpec((BLK, D), lambda i: (i, 0)),
            pl.BlockSpec((BLK, 1), lambda i: (i, 0)),
            pl.BlockSpec((D, D), lambda i: (0, 0)),
            pl.BlockSpec((1, D), lambda i: (0, 0)),
        ],
        out_specs=pl.BlockSpec((BLK, D), lambda i: (i, 0)),
        out_shape=jax.ShapeDtypeStruct((B, D), jnp.float32),
    )(sums, length2d, w_pad, b_pad)


PERM = np.empty((D,), np.int32)
for _c in range(NG):
    for _j in range(LANES):
        PERM[32 * _c + _j] = 16 * _c + _j
        PERM[32 * _c + LANES + _j] = 64 + 16 * _c + _j


def kernel(x, length, embed_table, W, b):
    x_flat = x.reshape(-1)
    bits = lax.bitcast_convert_type(
        embed_table.astype(jnp.bfloat16), jnp.uint16)
    packed = bits[:, :DW].astype(jnp.uint32) | (
        bits[:, DW:].astype(jnp.uint32) << 16)
    table_i32 = lax.bitcast_convert_type(packed, jnp.int32)
    sums = _pool_sums(x_flat, table_i32)
    w_pad = jnp.zeros((D, D), jnp.float32).at[:, :OUT].set(W.T)
    w_perm = w_pad[PERM, :]
    b_pad = jnp.zeros((1, D), jnp.float32).at[0, :OUT].set(b)
    out = _head(sums, length.reshape(B, 1), w_perm, b_pad)
    return out[:, :OUT]


# speedup vs baseline: 2.5812x; 1.0324x over previous
import dataclasses
import functools

import jax
import jax.numpy as jnp
import numpy as np
from jax import lax
from jax.experimental import pallas as pl
from jax.experimental.pallas import tpu as pltpu
from jax.experimental.pallas import tpu_sc as plsc

B = 4096
L = 200
D = 128
DW = D // 2
VOCAB = 100000
OUT = 2
NC = 2
NS = 16
NW = NC * NS
BPW = B // NW
SP1 = 104
SP2 = L - SP1
LANES = 16
NG = DW // LANES
UNROLL = 8


def _pack_table(table):
    ROWS = 800

    def body(t_ref, o_ref):
        u = lax.bitcast_convert_type(
            t_ref[...].astype(jnp.bfloat16), jnp.uint16)
        v = u.reshape(ROWS // 2, 2 * D)
        def pack(lo, hi):
            return lo.astype(jnp.uint32) | (hi.astype(jnp.uint32) << 16)
        w_even = pack(v[:, 0:DW], v[:, DW:D])
        w_odd = pack(v[:, D:D + DW], v[:, D + DW:2 * D])
        out2 = jnp.concatenate([w_even, w_odd], axis=1)
        o_ref[...] = lax.bitcast_convert_type(out2, jnp.int32).reshape(-1)

    return pl.pallas_call(
        body,
        grid=(VOCAB // ROWS,),
        in_specs=[pl.BlockSpec((ROWS, D), lambda i: (i, 0))],
        out_specs=pl.BlockSpec((ROWS * DW,), lambda i: (i,)),
        out_shape=jax.ShapeDtypeStruct((VOCAB * DW,), jnp.int32),
    )(table)


def _pool_sums(x_flat, table_i32):
    mesh = plsc.VectorSubcoreMesh(core_axis_name="c", subcore_axis_name="s")
    cp = pltpu.CompilerParams()
    if "needs_layout_passes" in pltpu.CompilerParams.__dataclass_fields__:
        cp = dataclasses.replace(cp, needs_layout_passes=False)
    if "use_tc_tiling_on_sc" in pltpu.CompilerParams.__dataclass_fields__:
        cp = dataclasses.replace(cp, use_tc_tiling_on_sc=False)

    @functools.partial(
        pl.kernel,
        out_type=jax.ShapeDtypeStruct((B * D,), jnp.float32),
        mesh=mesh,
        compiler_params=cp,
        scratch_types=[
            pltpu.VMEM((BPW * L,), jnp.int32),
            pltpu.VMEM((2, L, DW), jnp.int32),
            pltpu.VMEM((BPW * D,), jnp.float32),
            pltpu.SemaphoreType.DMA,
            pltpu.SemaphoreType.DMA,
        ],
    )
    def k(x_hbm, table_hbm, out_hbm, idx_v, rows_v, acc_v, sem0, sem1):
        wid = lax.axis_index("s") * NC + lax.axis_index("c")
        base = wid * BPW
        pltpu.sync_copy(x_hbm.at[pl.ds(base * L, BPW * L)], idx_v)
        sems = (sem0, sem1)
        mask_hi = jnp.full((LANES,), -65536, jnp.int32)

        def start(r, buf):
            off = r * L
            pltpu.async_copy(
                table_hbm.at[idx_v.at[pl.ds(off, SP1)]],
                rows_v.at[buf, pl.ds(0, SP1)], sems[buf])
            pltpu.async_copy(
                table_hbm.at[idx_v.at[pl.ds(off + SP1, SP2)]],
                rows_v.at[buf, pl.ds(SP1, SP2)], sems[buf])

        def wait(buf):
            pltpu.make_async_copy(
                table_hbm.at[pl.ds(0, L)], rows_v.at[buf], sems[buf]).wait()

        def process(r, buf):
            rv = rows_v.at[buf]

            def body(i, accs):
                t0 = i * UNROLL
                for u in range(UNROLL):
                    new = []
                    for c in range(NG):
                        w = rv[t0 + u, pl.ds(c * LANES, LANES)]
                        lo = plsc.bitcast(w << 16, jnp.float32)
                        hi = plsc.bitcast(w & mask_hi, jnp.float32)
                        new.append(accs[2 * c] + lo)
                        new.append(accs[2 * c + 1] + hi)
                    accs = tuple(new)
                return accs

            accs = lax.fori_loop(
                0, L // UNROLL, body,
                tuple(jnp.zeros((LANES,), jnp.float32) for _ in range(2 * NG)))
            for c in range(NG):
                acc_v[pl.ds(r * D + 2 * c * LANES, LANES)] = accs[2 * c]
                acc_v[pl.ds(r * D + (2 * c + 1) * LANES, LANES)] = (
                    accs[2 * c + 1])

        start(0, 0)
        start(1, 1)

        @pl.loop(0, BPW - 2, step=2)
        def _(i):
            wait(0)
            process(i, 0)
            start(i + 2, 0)
            wait(1)
            process(i + 1, 1)
            start(i + 3, 1)

        wait(0)
        process(BPW - 2, 0)
        wait(1)
        process(BPW - 1, 1)

        pltpu.sync_copy(acc_v, out_hbm.at[pl.ds(base * D, BPW * D)])

    return k(x_flat, table_i32)


def _head(sums, length2d, w_pad, b_pad):
    BLK = 512

    def body(p_ref, l_ref, w_ref, b_ref, o_ref):
        p = p_ref[...] / l_ref[...]
        z = jnp.dot(p, w_ref[...], preferred_element_type=jnp.float32)
        o_ref[...] = 1.0 / (1.0 + jnp.exp(-(z + b_ref[...])))

    return pl.pallas_call(
        body,
        grid=(B // BLK,),
        in_specs=[
            pl.BlockSpec((BLK, D), lambda i: (i, 0)),
            pl.BlockSpec((BLK, 1), lambda i: (i, 0)),
            pl.BlockSpec((D, D), lambda i: (0, 0)),
            pl.BlockSpec((1, D), lambda i: (0, 0)),
        ],
        out_specs=pl.BlockSpec((BLK, D), lambda i: (i, 0)),
        out_shape=jax.ShapeDtypeStruct((B, D), jnp.float32),
    )(sums, length2d, w_pad, b_pad)


PERM = np.empty((D,), np.int32)
for _c in range(NG):
    for _j in range(LANES):
        PERM[32 * _c + _j] = 16 * _c + _j
        PERM[32 * _c + LANES + _j] = 64 + 16 * _c + _j


def kernel(x, length, embed_table, W, b):
    x_flat = x.reshape(-1)
    table_i32 = _pack_table(embed_table).reshape(VOCAB, DW)
    sums = _pool_sums(x_flat, table_i32).reshape(B, D)
    w_pad = jnp.zeros((D, D), jnp.float32).at[:, :OUT].set(W.T)
    w_perm = w_pad[PERM, :]
    b_pad = jnp.zeros((1, D), jnp.float32).at[0, :OUT].set(b)
    out = _head(sums, length.reshape(B, 1), w_perm, b_pad)
    return out[:, :OUT]


# speedup vs baseline: 3.2385x; 1.2547x over previous
import dataclasses
import functools

import jax
import jax.numpy as jnp
import numpy as np
from jax import lax
from jax.experimental import pallas as pl
from jax.experimental.pallas import tpu as pltpu
from jax.experimental.pallas import tpu_sc as plsc

B = 4096
L = 200
D = 128
DW = D // 2
VOCAB = 100000
OUT = 2
NC = 2
NS = 16
NW = NC * NS
BPW = B // NW
SP1 = 104
SP2 = L - SP1
LANES = 16
NG = DW // LANES
UNROLL = 8


def _pack_table(table):
    ROWS = 4000

    def body(t_ref, o_ref):
        u = lax.bitcast_convert_type(
            t_ref[...].astype(jnp.bfloat16), jnp.uint16)
        v = u.reshape(ROWS // 2, 2 * D)
        def pack(lo, hi):
            return lo.astype(jnp.uint32) | (hi.astype(jnp.uint32) << 16)
        w_even = pack(v[:, 0:DW], v[:, DW:D])
        w_odd = pack(v[:, D:D + DW], v[:, D + DW:2 * D])
        out2 = jnp.concatenate([w_even, w_odd], axis=1)
        o_ref[...] = lax.bitcast_convert_type(out2, jnp.int32).reshape(-1)

    return pl.pallas_call(
        body,
        grid=(VOCAB // ROWS,),
        in_specs=[pl.BlockSpec((ROWS, D), lambda i: (i, 0))],
        out_specs=pl.BlockSpec((ROWS * DW,), lambda i: (i,)),
        out_shape=jax.ShapeDtypeStruct((VOCAB * DW,), jnp.int32),
    )(table)


def _pool_sums(x_flat, table_i32):
    mesh = plsc.VectorSubcoreMesh(core_axis_name="c", subcore_axis_name="s")
    cp = pltpu.CompilerParams()
    if "needs_layout_passes" in pltpu.CompilerParams.__dataclass_fields__:
        cp = dataclasses.replace(cp, needs_layout_passes=False)
    if "use_tc_tiling_on_sc" in pltpu.CompilerParams.__dataclass_fields__:
        cp = dataclasses.replace(cp, use_tc_tiling_on_sc=False)

    @functools.partial(
        pl.kernel,
        out_type=jax.ShapeDtypeStruct((B * D,), jnp.float32),
        mesh=mesh,
        compiler_params=cp,
        scratch_types=[
            pltpu.VMEM((BPW * L,), jnp.int32),
            pltpu.VMEM((2, L, DW), jnp.int32),
            pltpu.VMEM((BPW * D,), jnp.float32),
            pltpu.SemaphoreType.DMA,
            pltpu.SemaphoreType.DMA,
        ],
    )
    def k(x_hbm, table_hbm, out_hbm, idx_v, rows_v, acc_v, sem0, sem1):
        wid = lax.axis_index("s") * NC + lax.axis_index("c")
        base = wid * BPW
        pltpu.sync_copy(x_hbm.at[pl.ds(base * L, BPW * L)], idx_v)
        sems = (sem0, sem1)
        mask_hi = jnp.full((LANES,), -65536, jnp.int32)

        def start(r, buf):
            off = r * L
            pltpu.async_copy(
                table_hbm.at[idx_v.at[pl.ds(off, SP1)]],
                rows_v.at[buf, pl.ds(0, SP1)], sems[buf])
            pltpu.async_copy(
                table_hbm.at[idx_v.at[pl.ds(off + SP1, SP2)]],
                rows_v.at[buf, pl.ds(SP1, SP2)], sems[buf])

        def wait(buf):
            pltpu.make_async_copy(
                table_hbm.at[pl.ds(0, L)], rows_v.at[buf], sems[buf]).wait()

        def process(r, buf):
            rv = rows_v.at[buf]

            def body(i, accs):
                t0 = i * UNROLL
                for u in range(UNROLL):
                    new = []
                    for c in range(NG):
                        w = rv[t0 + u, pl.ds(c * LANES, LANES)]
                        lo = plsc.bitcast(w << 16, jnp.float32)
                        hi = plsc.bitcast(w & mask_hi, jnp.float32)
                        new.append(accs[2 * c] + lo)
                        new.append(accs[2 * c + 1] + hi)
                    accs = tuple(new)
                return accs

            accs = lax.fori_loop(
                0, L // UNROLL, body,
                tuple(jnp.zeros((LANES,), jnp.float32) for _ in range(2 * NG)))
            for c in range(NG):
                acc_v[pl.ds(r * D + 2 * c * LANES, LANES)] = accs[2 * c]
                acc_v[pl.ds(r * D + (2 * c + 1) * LANES, LANES)] = (
                    accs[2 * c + 1])

        start(0, 0)
        start(1, 1)

        @pl.loop(0, BPW - 2, step=2)
        def _(i):
            wait(0)
            process(i, 0)
            start(i + 2, 0)
            wait(1)
            process(i + 1, 1)
            start(i + 3, 1)

        wait(0)
        process(BPW - 2, 0)
        wait(1)
        process(BPW - 1, 1)

        pltpu.sync_copy(acc_v, out_hbm.at[pl.ds(base * D, BPW * D)])

    return k(x_flat, table_i32)


def _head(sums, length2d, w_pad, b_pad):
    BLK = 512

    def body(p_ref, l_ref, w_ref, b_ref, o_ref):
        p = p_ref[...] / l_ref[...]
        z = jnp.dot(p, w_ref[...], preferred_element_type=jnp.float32)
        o_ref[...] = 1.0 / (1.0 + jnp.exp(-(z + b_ref[...])))

    return pl.pallas_call(
        body,
        grid=(B // BLK,),
        in_specs=[
            pl.BlockSpec((BLK, D), lambda i: (i, 0)),
            pl.BlockSpec((BLK, 1), lambda i: (i, 0)),
            pl.BlockSpec((D, D), lambda i: (0, 0)),
            pl.BlockSpec((1, D), lambda i: (0, 0)),
        ],
        out_specs=pl.BlockSpec((BLK, D), lambda i: (i, 0)),
        out_shape=jax.ShapeDtypeStruct((B, D), jnp.float32),
    )(sums, length2d, w_pad, b_pad)


PERM = np.empty((D,), np.int32)
for _c in range(NG):
    for _j in range(LANES):
        PERM[32 * _c + _j] = 16 * _c + _j
        PERM[32 * _c + LANES + _j] = 64 + 16 * _c + _j


def kernel(x, length, embed_table, W, b):
    x_flat = x.reshape(-1)
    table_i32 = _pack_table(embed_table).reshape(VOCAB, DW)
    sums = _pool_sums(x_flat, table_i32).reshape(B, D)
    w_pad = jnp.zeros((D, D), jnp.float32).at[:, :OUT].set(W.T)
    w_perm = w_pad[PERM, :]
    b_pad = jnp.zeros((1, D), jnp.float32).at[0, :OUT].set(b)
    out = _head(sums, length.reshape(B, 1), w_perm, b_pad)
    return out[:, :OUT]


# speedup vs baseline: 3.3806x; 1.0439x over previous
import dataclasses
import functools

import jax
import jax.numpy as jnp
import numpy as np
from jax import lax
from jax.experimental import pallas as pl
from jax.experimental.pallas import tpu as pltpu
from jax.experimental.pallas import tpu_sc as plsc

B = 4096
L = 200
D = 128
DW = D // 2
VOCAB = 100000
OUT = 2
NC = 2
NS = 16
NW = NC * NS
BPW = B // NW
SP1 = 104
SP2 = L - SP1
LANES = 16
NG = DW // LANES
UNROLL = 8


def _pack_table(table):
    ROWS = 20000

    def body(t_ref, o_ref):
        u = lax.bitcast_convert_type(
            t_ref[...].astype(jnp.bfloat16), jnp.uint16)
        v = u.reshape(ROWS // 2, 2 * D)
        def pack(lo, hi):
            return lo.astype(jnp.uint32) | (hi.astype(jnp.uint32) << 16)
        w_even = pack(v[:, 0:DW], v[:, DW:D])
        w_odd = pack(v[:, D:D + DW], v[:, D + DW:2 * D])
        out2 = jnp.concatenate([w_even, w_odd], axis=1)
        o_ref[...] = lax.bitcast_convert_type(out2, jnp.int32).reshape(-1)

    return pl.pallas_call(
        body,
        grid=(VOCAB // ROWS,),
        in_specs=[pl.BlockSpec((ROWS, D), lambda i: (i, 0))],
        out_specs=pl.BlockSpec((ROWS * DW,), lambda i: (i,)),
        out_shape=jax.ShapeDtypeStruct((VOCAB * DW,), jnp.int32),
    )(table)


def _pool_sums(x_flat, table_i32):
    mesh = plsc.VectorSubcoreMesh(core_axis_name="c", subcore_axis_name="s")
    cp = pltpu.CompilerParams()
    if "needs_layout_passes" in pltpu.CompilerParams.__dataclass_fields__:
        cp = dataclasses.replace(cp, needs_layout_passes=False)
    if "use_tc_tiling_on_sc" in pltpu.CompilerParams.__dataclass_fields__:
        cp = dataclasses.replace(cp, use_tc_tiling_on_sc=False)

    @functools.partial(
        pl.kernel,
        out_type=jax.ShapeDtypeStruct((B * D,), jnp.float32),
        mesh=mesh,
        compiler_params=cp,
        scratch_types=[
            pltpu.VMEM((BPW * L,), jnp.int32),
            pltpu.VMEM((2, L, DW), jnp.int32),
            pltpu.VMEM((BPW * D,), jnp.float32),
            pltpu.SemaphoreType.DMA,
            pltpu.SemaphoreType.DMA,
        ],
    )
    def k(x_hbm, table_hbm, out_hbm, idx_v, rows_v, acc_v, sem0, sem1):
        wid = lax.axis_index("s") * NC + lax.axis_index("c")
        base = wid * BPW
        pltpu.sync_copy(x_hbm.at[pl.ds(base * L, BPW * L)], idx_v)
        sems = (sem0, sem1)
        mask_hi = jnp.full((LANES,), -65536, jnp.int32)

        def start(r, buf):
            off = r * L
            pltpu.async_copy(
                table_hbm.at[idx_v.at[pl.ds(off, SP1)]],
                rows_v.at[buf, pl.ds(0, SP1)], sems[buf])
            pltpu.async_copy(
                table_hbm.at[idx_v.at[pl.ds(off + SP1, SP2)]],
                rows_v.at[buf, pl.ds(SP1, SP2)], sems[buf])

        def wait(buf):
            pltpu.make_async_copy(
                table_hbm.at[pl.ds(0, L)], rows_v.at[buf], sems[buf]).wait()

        def process(r, buf):
            rv = rows_v.at[buf]

            def body(i, accs):
                t0 = i * UNROLL
                for u in range(UNROLL):
                    new = []
                    for c in range(NG):
                        w = rv[t0 + u, pl.ds(c * LANES, LANES)]
                        lo = plsc.bitcast(w << 16, jnp.float32)
                        hi = plsc.bitcast(w & mask_hi, jnp.float32)
                        new.append(accs[2 * c] + lo)
                        new.append(accs[2 * c + 1] + hi)
                    accs = tuple(new)
                return accs

            accs = lax.fori_loop(
                0, L // UNROLL, body,
                tuple(jnp.zeros((LANES,), jnp.float32) for _ in range(2 * NG)))
            for c in range(NG):
                acc_v[pl.ds(r * D + 2 * c * LANES, LANES)] = accs[2 * c]
                acc_v[pl.ds(r * D + (2 * c + 1) * LANES, LANES)] = (
                    accs[2 * c + 1])

        start(0, 0)
        start(1, 1)

        @pl.loop(0, BPW - 2, step=2)
        def _(i):
            wait(0)
            process(i, 0)
            start(i + 2, 0)
            wait(1)
            process(i + 1, 1)
            start(i + 3, 1)

        wait(0)
        process(BPW - 2, 0)
        wait(1)
        process(BPW - 1, 1)

        pltpu.sync_copy(acc_v, out_hbm.at[pl.ds(base * D, BPW * D)])

    return k(x_flat, table_i32)


def _head(sums, length2d, w_pad, b_pad):
    BLK = 512

    def body(p_ref, l_ref, w_ref, b_ref, o_ref):
        p = p_ref[...] / l_ref[...]
        z = jnp.dot(p, w_ref[...], preferred_element_type=jnp.float32)
        o_ref[...] = 1.0 / (1.0 + jnp.exp(-(z + b_ref[...])))

    return pl.pallas_call(
        body,
        grid=(B // BLK,),
        in_specs=[
            pl.BlockSpec((BLK, D), lambda i: (i, 0)),
            pl.BlockSpec((BLK, 1), lambda i: (i, 0)),
            pl.BlockSpec((D, D), lambda i: (0, 0)),
            pl.BlockSpec((1, D), lambda i: (0, 0)),
        ],
        out_specs=pl.BlockSpec((BLK, D), lambda i: (i, 0)),
        out_shape=jax.ShapeDtypeStruct((B, D), jnp.float32),
    )(sums, length2d, w_pad, b_pad)


PERM = np.empty((D,), np.int32)
for _c in range(NG):
    for _j in range(LANES):
        PERM[32 * _c + _j] = 16 * _c + _j
        PERM[32 * _c + LANES + _j] = 64 + 16 * _c + _j


def kernel(x, length, embed_table, W, b):
    x_flat = x.reshape(-1)
    table_i32 = _pack_table(embed_table).reshape(VOCAB, DW)
    sums = _pool_sums(x_flat, table_i32).reshape(B, D)
    w_pad = jnp.zeros((D, D), jnp.float32).at[:, :OUT].set(W.T)
    w_perm = w_pad[PERM, :]
    b_pad = jnp.zeros((1, D), jnp.float32).at[0, :OUT].set(b)
    out = _head(sums, length.reshape(B, 1), w_perm, b_pad)
    return out[:, :OUT]


# speedup vs baseline: 4.1151x; 1.2173x over previous
import dataclasses
import functools

import jax
import jax.numpy as jnp
import numpy as np
from jax import lax
from jax.experimental import pallas as pl
from jax.experimental.pallas import tpu as pltpu
from jax.experimental.pallas import tpu_sc as plsc

B = 4096
L = 200
D = 128
DW = D // 2
VOCAB = 100000
OUT = 2
NC = 2
NS = 16
NW = NC * NS
BPW = B // NW
SP1 = 104
SP2 = L - SP1
LANES = 16
NG = DW // LANES
UNROLL = 8


def _pack_table(table):
    ROWS = 20000

    def body(t_ref, o_ref):
        u = lax.bitcast_convert_type(
            t_ref[...].astype(jnp.bfloat16), jnp.uint16)
        v = u.reshape(ROWS // 2, 2 * D)
        def pack(lo, hi):
            return lo.astype(jnp.uint32) | (hi.astype(jnp.uint32) << 16)
        w_even = pack(v[:, 0:DW], v[:, DW:D])
        w_odd = pack(v[:, D:D + DW], v[:, D + DW:2 * D])
        out2 = jnp.concatenate([w_even, w_odd], axis=1)
        o_ref[...] = lax.bitcast_convert_type(out2, jnp.int32).reshape(-1)

    return pl.pallas_call(
        body,
        grid=(VOCAB // ROWS,),
        in_specs=[pl.BlockSpec((ROWS, D), lambda i: (i, 0))],
        out_specs=pl.BlockSpec((ROWS * DW,), lambda i: (i,)),
        out_shape=jax.ShapeDtypeStruct((VOCAB * DW,), jnp.int32),
    )(table)


def _pool_sums(x_flat, table_i32):
    mesh = plsc.VectorSubcoreMesh(core_axis_name="c", subcore_axis_name="s")
    cp = pltpu.CompilerParams()
    if "needs_layout_passes" in pltpu.CompilerParams.__dataclass_fields__:
        cp = dataclasses.replace(cp, needs_layout_passes=False)
    if "use_tc_tiling_on_sc" in pltpu.CompilerParams.__dataclass_fields__:
        cp = dataclasses.replace(cp, use_tc_tiling_on_sc=False)

    @functools.partial(
        pl.kernel,
        out_type=jax.ShapeDtypeStruct((B * D,), jnp.float32),
        mesh=mesh,
        compiler_params=cp,
        scratch_types=[
            pltpu.VMEM((BPW * L,), jnp.int32),
            pltpu.VMEM((4, L, DW), jnp.int32),
            pltpu.VMEM((BPW * D,), jnp.float32),
            pltpu.SemaphoreType.DMA,
            pltpu.SemaphoreType.DMA,
            pltpu.SemaphoreType.DMA,
            pltpu.SemaphoreType.DMA,
        ],
    )
    def k(x_hbm, table_hbm, out_hbm, idx_v, rows_v, acc_v,
          sem0, sem1, sem2, sem3):
        wid = lax.axis_index("s") * NC + lax.axis_index("c")
        base = wid * BPW
        pltpu.sync_copy(x_hbm.at[pl.ds(base * L, BPW * L)], idx_v)
        sems = (sem0, sem1, sem2, sem3)
        mask_hi = jnp.full((LANES,), -65536, jnp.int32)

        def start(r, buf):
            off = r * L
            pltpu.async_copy(
                table_hbm.at[idx_v.at[pl.ds(off, SP1)]],
                rows_v.at[buf, pl.ds(0, SP1)], sems[buf])
            pltpu.async_copy(
                table_hbm.at[idx_v.at[pl.ds(off + SP1, SP2)]],
                rows_v.at[buf, pl.ds(SP1, SP2)], sems[buf])

        def wait(buf):
            pltpu.make_async_copy(
                table_hbm.at[pl.ds(0, L)], rows_v.at[buf], sems[buf]).wait()

        def process(r, buf):
            rv = rows_v.at[buf]

            def body(i, accs):
                t0 = i * UNROLL
                for u in range(UNROLL):
                    new = []
                    for c in range(NG):
                        w = rv[t0 + u, pl.ds(c * LANES, LANES)]
                        lo = plsc.bitcast(w << 16, jnp.float32)
                        hi = plsc.bitcast(w & mask_hi, jnp.float32)
                        new.append(accs[2 * c] + lo)
                        new.append(accs[2 * c + 1] + hi)
                    accs = tuple(new)
                return accs

            accs = lax.fori_loop(
                0, L // UNROLL, body,
                tuple(jnp.zeros((LANES,), jnp.float32) for _ in range(2 * NG)))
            for c in range(NG):
                acc_v[pl.ds(r * D + 2 * c * LANES, LANES)] = accs[2 * c]
                acc_v[pl.ds(r * D + (2 * c + 1) * LANES, LANES)] = (
                    accs[2 * c + 1])

        NBUF = 4
        for b in range(NBUF):
            start(b, b)

        @pl.loop(0, BPW - NBUF, step=NBUF)
        def _(i):
            for b in range(NBUF):
                wait(b)
                process(i + b, b)
                start(i + NBUF + b, b)

        for b in range(NBUF):
            wait(b)
            process(BPW - NBUF + b, b)

        pltpu.sync_copy(acc_v, out_hbm.at[pl.ds(base * D, BPW * D)])

    return k(x_flat, table_i32)


def _head(sums, length2d, w_pad, b_pad):
    BLK = 512

    def body(p_ref, l_ref, w_ref, b_ref, o_ref):
        p = p_ref[...] / l_ref[...]
        z = jnp.dot(p, w_ref[...], preferred_element_type=jnp.float32)
        o_ref[...] = 1.0 / (1.0 + jnp.exp(-(z + b_ref[...])))

    return pl.pallas_call(
        body,
        grid=(B // BLK,),
        in_specs=[
            pl.BlockSpec((BLK, D), lambda i: (i, 0)),
            pl.BlockSpec((BLK, 1), lambda i: (i, 0)),
            pl.BlockSpec((D, D), lambda i: (0, 0)),
            pl.BlockSpec((1, D), lambda i: (0, 0)),
        ],
        out_specs=pl.BlockSpec((BLK, D), lambda i: (i, 0)),
        out_shape=jax.ShapeDtypeStruct((B, D), jnp.float32),
    )(sums, length2d, w_pad, b_pad)


PERM = np.empty((D,), np.int32)
for _c in range(NG):
    for _j in range(LANES):
        PERM[32 * _c + _j] = 16 * _c + _j
        PERM[32 * _c + LANES + _j] = 64 + 16 * _c + _j


def kernel(x, length, embed_table, W, b):
    x_flat = x.reshape(-1)
    table_i32 = _pack_table(embed_table).reshape(VOCAB, DW)
    sums = _pool_sums(x_flat, table_i32).reshape(B, D)
    w_pad = jnp.zeros((D, D), jnp.float32).at[:, :OUT].set(W.T)
    w_perm = w_pad[PERM, :]
    b_pad = jnp.zeros((1, D), jnp.float32).at[0, :OUT].set(b)
    out = _head(sums, length.reshape(B, 1), w_perm, b_pad)
    return out[:, :OUT]


# speedup vs baseline: 4.5009x; 1.0937x over previous
import dataclasses
import functools

import jax
import jax.numpy as jnp
import numpy as np
from jax import lax
from jax.experimental import pallas as pl
from jax.experimental.pallas import tpu as pltpu
from jax.experimental.pallas import tpu_sc as plsc

B = 4096
L = 200
D = 128
DW = D // 2
VOCAB = 100000
OUT = 2
NC = 2
NS = 16
NW = NC * NS
BPW = B // NW
SP1 = 104
SP2 = L - SP1
LANES = 16
NG = DW // LANES
UNROLL = 8


def _pack_table(table):
    ROWS = 20000

    def body(t_ref, o_ref):
        u = lax.bitcast_convert_type(
            t_ref[...].astype(jnp.bfloat16), jnp.uint16)
        v = u.reshape(ROWS // 2, 2 * D)
        def pack(lo, hi):
            return lo.astype(jnp.uint32) | (hi.astype(jnp.uint32) << 16)
        w_even = pack(v[:, 0:DW], v[:, DW:D])
        w_odd = pack(v[:, D:D + DW], v[:, D + DW:2 * D])
        out2 = jnp.concatenate([w_even, w_odd], axis=1)
        o_ref[...] = lax.bitcast_convert_type(out2, jnp.int32).reshape(-1)

    return pl.pallas_call(
        body,
        grid=(VOCAB // ROWS,),
        in_specs=[pl.BlockSpec((ROWS, D), lambda i: (i, 0))],
        out_specs=pl.BlockSpec((ROWS * DW,), lambda i: (i,)),
        out_shape=jax.ShapeDtypeStruct((VOCAB * DW,), jnp.int32),
    )(table)


def _pool_sums(x_flat, table_i32):
    mesh = plsc.VectorSubcoreMesh(core_axis_name="c", subcore_axis_name="s")
    cp = pltpu.CompilerParams()
    if "needs_layout_passes" in pltpu.CompilerParams.__dataclass_fields__:
        cp = dataclasses.replace(cp, needs_layout_passes=False)
    if "use_tc_tiling_on_sc" in pltpu.CompilerParams.__dataclass_fields__:
        cp = dataclasses.replace(cp, use_tc_tiling_on_sc=False)

    @functools.partial(
        pl.kernel,
        out_type=jax.ShapeDtypeStruct((B * D,), jnp.float32),
        mesh=mesh,
        compiler_params=cp,
        scratch_types=[
            pltpu.VMEM((BPW * L,), jnp.int32),
            pltpu.VMEM((4, L, DW), jnp.int32),
            pltpu.VMEM((BPW * D,), jnp.float32),
            pltpu.SemaphoreType.DMA,
            pltpu.SemaphoreType.DMA,
            pltpu.SemaphoreType.DMA,
            pltpu.SemaphoreType.DMA,
        ],
    )
    def k(x_hbm, table_hbm, out_hbm, idx_v, rows_v, acc_v,
          sem0, sem1, sem2, sem3):
        wid = lax.axis_index("s") * NC + lax.axis_index("c")
        base = wid * BPW
        pltpu.sync_copy(x_hbm.at[pl.ds(base * L, BPW * L)], idx_v)
        sems = (sem0, sem1, sem2, sem3)
        mask_hi = jnp.full((LANES,), -65536, jnp.int32)

        def start(r, buf):
            off = r * L
            pltpu.async_copy(
                table_hbm.at[idx_v.at[pl.ds(off, SP1)]],
                rows_v.at[buf, pl.ds(0, SP1)], sems[buf])
            pltpu.async_copy(
                table_hbm.at[idx_v.at[pl.ds(off + SP1, SP2)]],
                rows_v.at[buf, pl.ds(SP1, SP2)], sems[buf])

        def wait(buf):
            pltpu.make_async_copy(
                table_hbm.at[pl.ds(0, L)], rows_v.at[buf], sems[buf]).wait()

        def process(r, buf):
            rv = rows_v.at[buf]

            def body(i, accs):
                t0 = i * UNROLL
                for u in range(UNROLL):
                    accs = tuple(
                        accs[c] + plsc.bitcast(
                            rv[t0 + u, pl.ds(c * LANES, LANES)],
                            jnp.bfloat16)
                        for c in range(NG))
                return accs

            accs = lax.fori_loop(
                0, L // UNROLL, body,
                tuple(jnp.zeros((2 * LANES,), jnp.bfloat16)
                      for _ in range(NG)))
            for c in range(NG):
                w = plsc.bitcast(accs[c], jnp.int32)
                lo = plsc.bitcast(w << 16, jnp.float32)
                hi = plsc.bitcast(w & mask_hi, jnp.float32)
                acc_v[pl.ds(r * D + 2 * c * LANES, LANES)] = lo
                acc_v[pl.ds(r * D + (2 * c + 1) * LANES, LANES)] = hi

        NBUF = 4
        for b in range(NBUF):
            start(b, b)

        @pl.loop(0, BPW - NBUF, step=NBUF)
        def _(i):
            for b in range(NBUF):
                wait(b)
                process(i + b, b)
                start(i + NBUF + b, b)

        for b in range(NBUF):
            wait(b)
            process(BPW - NBUF + b, b)

        pltpu.sync_copy(acc_v, out_hbm.at[pl.ds(base * D, BPW * D)])

    return k(x_flat, table_i32)


def _head(sums, length2d, w_pad, b_pad):
    BLK = 512

    def body(p_ref, l_ref, w_ref, b_ref, o_ref):
        p = p_ref[...] / l_ref[...]
        z = jnp.dot(p, w_ref[...], preferred_element_type=jnp.float32)
        o_ref[...] = 1.0 / (1.0 + jnp.exp(-(z + b_ref[...])))

    return pl.pallas_call(
        body,
        grid=(B // BLK,),
        in_specs=[
            pl.BlockSpec((BLK, D), lambda i: (i, 0)),
            pl.BlockSpec((BLK, 1), lambda i: (i, 0)),
            pl.BlockSpec((D, D), lambda i: (0, 0)),
            pl.BlockSpec((1, D), lambda i: (0, 0)),
        ],
        out_specs=pl.BlockSpec((BLK, D), lambda i: (i, 0)),
        out_shape=jax.ShapeDtypeStruct((B, D), jnp.float32),
    )(sums, length2d, w_pad, b_pad)


PERM = np.empty((D,), np.int32)
for _c in range(NG):
    for _j in range(LANES):
        PERM[32 * _c + _j] = 16 * _c + _j
        PERM[32 * _c + LANES + _j] = 64 + 16 * _c + _j


def kernel(x, length, embed_table, W, b):
    x_flat = x.reshape(-1)
    table_i32 = _pack_table(embed_table).reshape(VOCAB, DW)
    sums = _pool_sums(x_flat, table_i32).reshape(B, D)
    w_pad = jnp.zeros((D, D), jnp.float32).at[:, :OUT].set(W.T)
    w_perm = w_pad[PERM, :]
    b_pad = jnp.zeros((1, D), jnp.float32).at[0, :OUT].set(b)
    out = _head(sums, length.reshape(B, 1), w_perm, b_pad)
    return out[:, :OUT]
